# baseline (speedup 1.0000x reference)
import functools

import jax
import jax.numpy as jnp
from jax.experimental import pallas as pl
from jax.experimental.pallas import tpu as pltpu

_IM = 64
_NC = 2
_NT = 16384
_TB = 8192


def _body(trj_ref, a_ref, out_r_ref, out_i_ref, e1s_ref):
    two_pi = 2.0 * jnp.pi
    th0 = (-two_pi) * (trj_ref[0:1, :] - 0.5)
    th1 = (-two_pi) * (trj_ref[1:2, :] - 0.5)
    w1r = jnp.cos(th1)
    w1i = jnp.sin(th1)
    pows1 = [(w1r, w1i)]
    for _ in range(5):
        pr, pi_ = pows1[-1]
        pows1.append((pr * pr - pi_ * pi_, 2.0 * pr * pi_))
    p32r1, p32i1 = pows1[5]
    er1, ei1 = p32r1, -p32i1
    for s in range(3):
        pr, pi_ = pows1[s]
        nr = er1 * pr - ei1 * pi_
        ni = er1 * pi_ + ei1 * pr
        er1 = jnp.concatenate([er1, nr], axis=0)
        ei1 = jnp.concatenate([ei1, ni], axis=0)
    e1s_ref[0:8, :] = er1
    e1s_ref[64:72, :] = ei1
    for s in (3, 4, 5):
        n = 1 << s
        pr, pi_ = pows1[s]
        rlo = e1s_ref[0:n, :]
        ilo = e1s_ref[64:64 + n, :]
        e1s_ref[n:2 * n, :] = rlo * pr - ilo * pi_
        e1s_ref[64 + n:64 + 2 * n, :] = rlo * pi_ + ilo * pr
    tmp = jnp.dot(a_ref[...], e1s_ref[...],
                  preferred_element_type=jnp.float32)

    wr = jnp.cos(th0)
    wi = jnp.sin(th0)
    pows = [(wr, wi)]
    for _ in range(5):
        pr, pi_ = pows[-1]
        pows.append((pr * pr - pi_ * pi_, 2.0 * pr * pi_))
    er = jnp.concatenate([jnp.ones_like(wr), wr], axis=0)
    ei = jnp.concatenate([jnp.zeros_like(wi), wi], axis=0)
    for s in (1, 2):
        pr, pi_ = pows[s]
        er, ei = (
            jnp.concatenate([er, er * pr - ei * pi_], axis=0),
            jnp.concatenate([ei, er * pi_ + ei * pr], axis=0),
        )
    p32r, p32i = pows[5]

    outs_r = []
    outs_i = []
    for c in range(_NC):
        ar = tmp[128 * c:128 * c + 64]
        ai = tmp[128 * c + 64:128 * c + 128]
        for s in (5, 4, 3):
            pr, pi_ = pows[s]
            half = 1 << s
            lo_r, hi_r = ar[:half], ar[half:]
            lo_i, hi_i = ai[:half], ai[half:]
            ar = lo_r + (pr * hi_r - pi_ * hi_i)
            ai = lo_i + (pr * hi_i + pi_ * hi_r)
        sr = jnp.sum(er * ar - ei * ai, axis=0, keepdims=True)
        si = jnp.sum(er * ai + ei * ar, axis=0, keepdims=True)
        outs_r.append(sr * p32r + si * p32i)
        outs_i.append(si * p32r - sr * p32i)
    out_r_ref[...] = jnp.concatenate(outs_r, axis=0)
    out_i_ref[...] = jnp.concatenate(outs_i, axis=0)


@jax.jit
def _nufft(img_real, img_imag, trj):
    ir = img_real[0]
    ii = img_imag[0]

    def coil_block(c):
        return jnp.concatenate(
            [
                jnp.concatenate([ir[c], -ii[c]], axis=1),
                jnp.concatenate([ii[c], ir[c]], axis=1),
            ],
            axis=0,
        )

    a = jnp.concatenate([coil_block(0), coil_block(1)], axis=0)
    trj_t = trj[0].T

    grid = (_NT // _TB,)
    out_r, out_i = pl.pallas_call(
        _body,
        grid=grid,
        in_specs=[
            pl.BlockSpec((2, _TB), lambda i: (0, i)),
            pl.BlockSpec((256, 128), lambda i: (0, 0)),
        ],
        out_specs=[
            pl.BlockSpec((_NC, _TB), lambda i: (0, i)),
            pl.BlockSpec((_NC, _TB), lambda i: (0, i)),
        ],
        out_shape=[
            jax.ShapeDtypeStruct((_NC, _NT), jnp.float32),
            jax.ShapeDtypeStruct((_NC, _NT), jnp.float32),
        ],
        scratch_shapes=[pltpu.VMEM((2 * _IM, _TB), jnp.float32)],
    )(trj_t, a)
    return jax.lax.complex(out_r, out_i)[None]


def kernel(img_real, img_imag, trj):
    return _nufft(img_real, img_imag, trj)


# speedup vs baseline: 1.1402x; 1.1402x over previous
import functools

import jax
import jax.numpy as jnp
from jax.experimental import pallas as pl
from jax.experimental.pallas import tpu as pltpu

_IM = 64
_NC = 2
_NT = 16384
_TB = 8192


def _body(trj_ref, ir_ref, ii_ref, out_r_ref, out_i_ref, e1s_ref):
    two_pi = 2.0 * jnp.pi
    th0 = (-two_pi) * (trj_ref[0:1, :] - 0.5)
    th1 = (-two_pi) * (trj_ref[1:2, :] - 0.5)
    w1r = jnp.cos(th1)
    w1i = jnp.sin(th1)
    pows1 = [(w1r, w1i)]
    for _ in range(5):
        pr, pi_ = pows1[-1]
        pows1.append((pr * pr - pi_ * pi_, 2.0 * pr * pi_))
    p32r1, p32i1 = pows1[5]
    er1, ei1 = p32r1, -p32i1
    for s in range(3):
        pr, pi_ = pows1[s]
        nr = er1 * pr - ei1 * pi_
        ni = er1 * pi_ + ei1 * pr
        er1 = jnp.concatenate([er1, nr], axis=0)
        ei1 = jnp.concatenate([ei1, ni], axis=0)
    e1s_ref[0:8, :] = er1
    e1s_ref[64:72, :] = ei1
    for s in (3, 4, 5):
        n = 1 << s
        pr, pi_ = pows1[s]
        rlo = e1s_ref[0:n, :]
        ilo = e1s_ref[64:64 + n, :]
        e1s_ref[n:2 * n, :] = rlo * pr - ilo * pi_
        e1s_ref[64 + n:64 + 2 * n, :] = rlo * pi_ + ilo * pr
    i0r, i0i, i1r, i1i = ir_ref[0], ii_ref[0], ir_ref[1], ii_ref[1]
    a = jnp.concatenate(
        [
            jnp.concatenate([i0r, -i0i], axis=1),
            jnp.concatenate([i0i, i0r], axis=1),
            jnp.concatenate([i1r, -i1i], axis=1),
            jnp.concatenate([i1i, i1r], axis=1),
        ],
        axis=0,
    )
    tmp = jnp.dot(a, e1s_ref[...],
                  preferred_element_type=jnp.float32)

    wr = jnp.cos(th0)
    wi = jnp.sin(th0)
    pows = [(wr, wi)]
    for _ in range(5):
        pr, pi_ = pows[-1]
        pows.append((pr * pr - pi_ * pi_, 2.0 * pr * pi_))
    er = jnp.concatenate([jnp.ones_like(wr), wr], axis=0)
    ei = jnp.concatenate([jnp.zeros_like(wi), wi], axis=0)
    for s in (1, 2):
        pr, pi_ = pows[s]
        er, ei = (
            jnp.concatenate([er, er * pr - ei * pi_], axis=0),
            jnp.concatenate([ei, er * pi_ + ei * pr], axis=0),
        )
    p32r, p32i = pows[5]

    outs_r = []
    outs_i = []
    for c in range(_NC):
        ar = tmp[128 * c:128 * c + 64]
        ai = tmp[128 * c + 64:128 * c + 128]
        for s in (5, 4, 3):
            pr, pi_ = pows[s]
            half = 1 << s
            lo_r, hi_r = ar[:half], ar[half:]
            lo_i, hi_i = ai[:half], ai[half:]
            ar = lo_r + (pr * hi_r - pi_ * hi_i)
            ai = lo_i + (pr * hi_i + pi_ * hi_r)
        sr = jnp.sum(er * ar - ei * ai, axis=0, keepdims=True)
        si = jnp.sum(er * ai + ei * ar, axis=0, keepdims=True)
        outs_r.append(sr * p32r + si * p32i)
        outs_i.append(si * p32r - sr * p32i)
    out_r_ref[...] = jnp.concatenate(outs_r, axis=0)
    out_i_ref[...] = jnp.concatenate(outs_i, axis=0)


@jax.jit
def _nufft(img_real, img_imag, trj):
    trj_t = trj[0].T

    grid = (_NT // _TB,)
    out_r, out_i = pl.pallas_call(
        _body,
        grid=grid,
        in_specs=[
            pl.BlockSpec((2, _TB), lambda i: (0, i)),
            pl.BlockSpec((_NC, _IM, _IM), lambda i: (0, 0, 0)),
            pl.BlockSpec((_NC, _IM, _IM), lambda i: (0, 0, 0)),
        ],
        out_specs=[
            pl.BlockSpec((_NC, _TB), lambda i: (0, i)),
            pl.BlockSpec((_NC, _TB), lambda i: (0, i)),
        ],
        out_shape=[
            jax.ShapeDtypeStruct((_NC, _NT), jnp.float32),
            jax.ShapeDtypeStruct((_NC, _NT), jnp.float32),
        ],
        scratch_shapes=[pltpu.VMEM((2 * _IM, _TB), jnp.float32)],
    )(trj_t, img_real[0], img_imag[0])
    return jax.lax.complex(out_r, out_i)[None]


def kernel(img_real, img_imag, trj):
    return _nufft(img_real, img_imag, trj)
